# baseline (speedup 1.0000x reference)
import functools

import jax
import jax.numpy as jnp
from jax import lax
from jax.experimental import pallas as pl
from jax.experimental.pallas import tpu as pltpu
from jax.experimental.pallas import tpu_sc as plsc

_LANES = 16
_CHUNK = 128
_CBLK = 16384


def _tc_scores_body(tu_ref, tp_ref, w_ref, b_ref, su_ref, sp_ref):
    d = tu_ref.shape[0]
    wu = jnp.broadcast_to(w_ref[0:d, 0], (8, d))
    wp = jnp.broadcast_to(w_ref[d:2 * d, 0], (8, d))
    b = b_ref[0]
    su_ref[...] = jnp.dot(wu, tu_ref[...],
                          preferred_element_type=jnp.float32)[0] + b
    sp_ref[...] = jnp.dot(wp, tp_ref[...],
                          preferred_element_type=jnp.float32)[0]


@functools.lru_cache(maxsize=None)
def _make_tc_scores(n_rows, d):
    grid = (n_rows + _CBLK - 1) // _CBLK
    return pl.pallas_call(
        _tc_scores_body,
        grid=(grid,),
        in_specs=[
            pl.BlockSpec((d, _CBLK), lambda i: (0, i)),
            pl.BlockSpec((d, _CBLK), lambda i: (0, i)),
            pl.BlockSpec((2 * d, 1), lambda i: (0, 0)),
            pl.BlockSpec(memory_space=pltpu.SMEM),
        ],
        out_specs=[
            pl.BlockSpec((_CBLK,), lambda i: (i,)),
            pl.BlockSpec((_CBLK,), lambda i: (i,)),
        ],
        out_shape=[
            jax.ShapeDtypeStruct((n_rows,), jnp.float32),
            jax.ShapeDtypeStruct((n_rows,), jnp.float32),
        ],
    )


@functools.lru_cache(maxsize=None)
def _make_sc_gather(B, n_cores, n_subcores):
    NW = n_cores * n_subcores
    per_w = B // NW
    n_chunks = per_w // _CHUNK
    mesh = plsc.VectorSubcoreMesh(core_axis_name="c", subcore_axis_name="s")

    @functools.partial(
        pl.kernel,
        out_type=jax.ShapeDtypeStruct((B,), jnp.float32),
        mesh=mesh,
        scratch_types=[
            pltpu.VMEM((per_w,), jnp.int32),
            pltpu.VMEM((per_w,), jnp.int32),
            pltpu.VMEM((per_w,), jnp.float32),
            pltpu.VMEM((per_w,), jnp.float32),
            pltpu.SemaphoreType.DMA,
        ],
        compiler_params=pltpu.CompilerParams(
            needs_layout_passes=False, use_tc_tiling_on_sc=True),
    )
    def sc_kernel(users_hbm, posts_hbm, su_hbm, sp_hbm, out_hbm,
                  idx_u, idx_p, vu, vp, sem):
        wid = lax.axis_index("s") * n_cores + lax.axis_index("c")
        base = wid * per_w
        pltpu.sync_copy(users_hbm.at[pl.ds(base, per_w)], idx_u)
        pltpu.sync_copy(posts_hbm.at[pl.ds(base, per_w)], idx_p)

        copies = []
        for k in range(n_chunks):
            sl = pl.ds(k * _CHUNK, _CHUNK)
            copies.append(
                pltpu.async_copy(su_hbm.at[idx_u.at[sl]], vu.at[sl], sem))
            copies.append(
                pltpu.async_copy(sp_hbm.at[idx_p.at[sl]], vp.at[sl], sem))
        for c in copies:
            c.wait()

        for g in range(per_w // _LANES):
            sl = pl.ds(g * _LANES, _LANES)
            vu[sl] = vu[sl] + vp[sl]
        pltpu.sync_copy(vu, out_hbm.at[pl.ds(base, per_w)])

    return sc_kernel


def kernel(users, posts, user_table, post_table, W, b):
    B = users.shape[0]
    n_rows, d = user_table.shape
    info = plsc.get_sparse_core_info()

    su, sp = _make_tc_scores(n_rows, d)(user_table.T, post_table.T, W, b)

    out = _make_sc_gather(B, info.num_cores, info.num_subcores)(
        users.astype(jnp.int32), posts.astype(jnp.int32), su, sp)
    return out.reshape(B, 1)


# speedup vs baseline: 1.1173x; 1.1173x over previous
import functools

import jax
import jax.numpy as jnp
from jax import lax
from jax.experimental import pallas as pl
from jax.experimental.pallas import tpu as pltpu
from jax.experimental.pallas import tpu_sc as plsc

_LANES = 16
_CHUNK = 128
_CBLK = 28672


def _tc_scores_body(tu_ref, tp_ref, w_ref, b_ref, su_ref, sp_ref):
    d = tu_ref.shape[0]
    wu = jnp.broadcast_to(w_ref[0:d, 0], (8, d))
    wp = jnp.broadcast_to(w_ref[d:2 * d, 0], (8, d))
    b = b_ref[0]
    su_ref[...] = jnp.dot(wu, tu_ref[...],
                          preferred_element_type=jnp.float32)[0] + b
    sp_ref[...] = jnp.dot(wp, tp_ref[...],
                          preferred_element_type=jnp.float32)[0]


@functools.lru_cache(maxsize=None)
def _make_tc_scores(n_rows, d):
    grid = (n_rows + _CBLK - 1) // _CBLK
    return pl.pallas_call(
        _tc_scores_body,
        grid=(grid,),
        in_specs=[
            pl.BlockSpec((d, _CBLK), lambda i: (0, i)),
            pl.BlockSpec((d, _CBLK), lambda i: (0, i)),
            pl.BlockSpec((2 * d, 1), lambda i: (0, 0)),
            pl.BlockSpec(memory_space=pltpu.SMEM),
        ],
        out_specs=[
            pl.BlockSpec((_CBLK,), lambda i: (i,)),
            pl.BlockSpec((_CBLK,), lambda i: (i,)),
        ],
        out_shape=[
            jax.ShapeDtypeStruct((n_rows,), jnp.float32),
            jax.ShapeDtypeStruct((n_rows,), jnp.float32),
        ],
    )


@functools.lru_cache(maxsize=None)
def _make_sc_gather(B, n_cores, n_subcores):
    NW = n_cores * n_subcores
    per_w = B // NW
    n_chunks = per_w // _CHUNK
    mesh = plsc.VectorSubcoreMesh(core_axis_name="c", subcore_axis_name="s")

    @functools.partial(
        pl.kernel,
        out_type=jax.ShapeDtypeStruct((B,), jnp.float32),
        mesh=mesh,
        scratch_types=[
            pltpu.VMEM((per_w,), jnp.int32),
            pltpu.VMEM((per_w,), jnp.int32),
            pltpu.VMEM((per_w,), jnp.float32),
            pltpu.VMEM((per_w,), jnp.float32),
            pltpu.SemaphoreType.DMA,
        ],
        compiler_params=pltpu.CompilerParams(
            needs_layout_passes=False, use_tc_tiling_on_sc=True),
    )
    def sc_kernel(users_hbm, posts_hbm, su_hbm, sp_hbm, out_hbm,
                  idx_u, idx_p, vu, vp, sem):
        wid = lax.axis_index("s") * n_cores + lax.axis_index("c")
        base = wid * per_w
        pltpu.sync_copy(users_hbm.at[pl.ds(base, per_w)], idx_u)
        pltpu.sync_copy(posts_hbm.at[pl.ds(base, per_w)], idx_p)

        copies = []
        for k in range(n_chunks):
            sl = pl.ds(k * _CHUNK, _CHUNK)
            copies.append(
                pltpu.async_copy(su_hbm.at[idx_u.at[sl]], vu.at[sl], sem))
            copies.append(
                pltpu.async_copy(sp_hbm.at[idx_p.at[sl]], vp.at[sl], sem))
        for c in copies:
            c.wait()

        for g in range(per_w // _LANES):
            sl = pl.ds(g * _LANES, _LANES)
            vu[sl] = vu[sl] + vp[sl]
        pltpu.sync_copy(vu, out_hbm.at[pl.ds(base, per_w)])

    return sc_kernel


def kernel(users, posts, user_table, post_table, W, b):
    B = users.shape[0]
    n_rows, d = user_table.shape
    info = plsc.get_sparse_core_info()

    su, sp = _make_tc_scores(n_rows, d)(user_table.T, post_table.T, W, b)

    out = _make_sc_gather(B, info.num_cores, info.num_subcores)(
        users.astype(jnp.int32), posts.astype(jnp.int32), su, sp)
    return out.reshape(B, 1)
